# baseline (speedup 1.0000x reference)
import functools

import jax
import jax.numpy as jnp
from jax import lax
from jax.experimental import pallas as pl
from jax.experimental.pallas import tpu as pltpu
from jax.experimental.pallas import tpu_sc as plsc

N_NODES = 10000
N_EDGES = 320000
D = 128

NC = 2
NS = 16
NW = NC * NS
E_PER_W = N_EDGES // NW
CHUNK = 128
FULL_CHUNKS = E_PER_W // CHUNK
TAIL_E = E_PER_W - FULL_CHUNKS * CHUNK
N_CH = FULL_CHUNKS + 1
E_PAD_W = N_CH * CHUNK
NB = 6
NR = 2
ROWS_PER_TILE = 624
TAIL_OFF = ROWS_PER_TILE * NS
TAIL_ROWS = N_NODES - TAIL_OFF


def _sc_scatter_sum(features, src_pad, dst, zeros):
    mesh = plsc.VectorSubcoreMesh(
        core_axis_name="c", subcore_axis_name="s", num_cores=NC, num_subcores=NS
    )

    @functools.partial(
        pl.kernel,
        out_type=jax.ShapeDtypeStruct((NC, N_NODES, D), jnp.float32),
        mesh=mesh,
        scratch_types=[
            pltpu.VMEM_SHARED((N_NODES, D), jnp.float32),
            [pltpu.VMEM((CHUNK,), jnp.int32)] * NB,
            [pltpu.VMEM((CHUNK,), jnp.int32)] * NB,
            pltpu.VMEM((TAIL_E,), jnp.int32),
            pltpu.VMEM((CHUNK, D), jnp.float32),
            pltpu.VMEM((CHUNK, D), jnp.float32),
            [pltpu.SemaphoreType.DMA] * NB,
            [pltpu.SemaphoreType.DMA] * NB,
            pltpu.SemaphoreType.DMA,
            pltpu.SemaphoreType.DMA,
            pltpu.SemaphoreType.DMA,
        ],
    )
    def k(feat_hbm, src_hbm, dst_hbm, zeros_hbm, out_hbm,
          acc, srcb, dstb, dbt, rows0, rows1, ssrc, sdst, sdt, sg0, sg1):
        c = lax.axis_index("c")
        s = lax.axis_index("s")
        wid = s * NC + c
        base_s = wid * E_PAD_W
        base_d = wid * E_PER_W
        rows = (rows0, rows1)
        sg = (sg0, sg1)

        pltpu.sync_copy(zeros_hbm, acc.at[pl.ds(s * ROWS_PER_TILE, ROWS_PER_TILE)])

        @pl.when(s == NS - 1)
        def _():
            pltpu.sync_copy(zeros_hbm.at[pl.ds(0, TAIL_ROWS)],
                            acc.at[pl.ds(TAIL_OFF, TAIL_ROWS)])

        for j in range(NB):
            pltpu.async_copy(src_hbm.at[pl.ds(base_s + j * CHUNK, CHUNK)],
                             srcb[j], ssrc[j])
            pltpu.async_copy(dst_hbm.at[pl.ds(base_d + j * CHUNK, CHUNK)],
                             dstb[j], sdst[j])
        pltpu.async_copy(
            dst_hbm.at[pl.ds(base_d + FULL_CHUNKS * CHUNK, TAIL_E)], dbt, sdt)
        plsc.subcore_barrier()

        idx_bytes = src_hbm.at[pl.ds(base_s, CHUNK)]

        for j in range(NR - 1):
            pltpu.make_async_copy(idx_bytes, srcb[j], ssrc[j]).wait()
            pltpu.async_copy(feat_hbm.at[srcb[j]], rows[j], sg[j])

        def step(i, b, bn, r, rn):
            pltpu.make_async_copy(idx_bytes, srcb[bn], ssrc[bn]).wait()
            pltpu.async_copy(feat_hbm.at[srcb[bn]], rows[rn], sg[rn])
            pltpu.make_async_copy(feat_hbm.at[srcb[b]], rows[r], sg[r]).wait()

            @pl.when(i + NB <= FULL_CHUNKS)
            def _():
                off = base_s + (i + NB) * CHUNK
                pltpu.async_copy(src_hbm.at[pl.ds(off, CHUNK)], srcb[b], ssrc[b])

            pltpu.make_async_copy(idx_bytes, dstb[b], sdst[b]).wait()
            pltpu.sync_copy(rows[r], acc.at[dstb[b]], add=True)

            @pl.when(i + NB < FULL_CHUNKS)
            def _():
                off = base_d + (i + NB) * CHUNK
                pltpu.async_copy(dst_hbm.at[pl.ds(off, CHUNK)], dstb[b], sdst[b])

        def body(t, _):
            i0 = t * NB
            for kk in range(NB):
                step(i0 + kk, kk, (kk + NR - 1) % NB, kk % NR, (kk + NR - 1) % NR)
            return ()

        lax.fori_loop(0, FULL_CHUNKS // NB, body, ())

        rt = FULL_CHUNKS % NR
        bt = FULL_CHUNKS % NB
        pltpu.make_async_copy(feat_hbm.at[srcb[bt]], rows[rt], sg[rt]).wait()
        pltpu.make_async_copy(
            dst_hbm.at[pl.ds(base_d, TAIL_E)], dbt, sdt).wait()
        pltpu.sync_copy(rows[rt].at[pl.ds(0, TAIL_E)], acc.at[dbt], add=True)

        plsc.subcore_barrier()
        pltpu.sync_copy(
            acc.at[pl.ds(s * ROWS_PER_TILE, ROWS_PER_TILE)],
            out_hbm.at[c, pl.ds(s * ROWS_PER_TILE, ROWS_PER_TILE)],
        )

        @pl.when(s == NS - 1)
        def _():
            pltpu.sync_copy(acc.at[pl.ds(TAIL_OFF, TAIL_ROWS)],
                            out_hbm.at[c, pl.ds(TAIL_OFF, TAIL_ROWS)])

    return k(features, src_pad, dst, zeros)


def _tc_feat_body(f_ref, wt_ref, b_ref, t_ref, o_ref):
    a = jnp.dot(f_ref[...], wt_ref[...], preferred_element_type=jnp.float32)
    o_ref[...] = t_ref[0, 0] * jnp.maximum(a + b_ref[...], 0.0)


def _tc_neigh_body(y1_ref, p0_ref, p1_ref, wt_ref, b_ref, t_ref, o_ref):
    hn = p0_ref[...] + p1_ref[...]
    a = jnp.dot(hn, wt_ref[...], preferred_element_type=jnp.float32)
    o_ref[...] = y1_ref[...] + t_ref[0, 0] * jnp.maximum(a + b_ref[...], 0.0)


_ROW_SPEC = pl.BlockSpec((1000, D), lambda i: (i, 0))
_FULL_SPECS = [
    pl.BlockSpec((D, D), lambda i: (0, 0)),
    pl.BlockSpec((1, D), lambda i: (0, 0)),
    pl.BlockSpec(memory_space=pltpu.SMEM),
]
_OUT_SHAPE = jax.ShapeDtypeStruct((N_NODES, D), jnp.float32)


def _tc_feat(features, wt, b2, t1):
    return pl.pallas_call(
        _tc_feat_body,
        grid=(10,),
        in_specs=[_ROW_SPEC] + _FULL_SPECS,
        out_specs=_ROW_SPEC,
        out_shape=_OUT_SHAPE,
    )(features, wt, b2, t1)


def _tc_neigh(y1, partials, wt, b2, t2):
    return pl.pallas_call(
        _tc_neigh_body,
        grid=(10,),
        in_specs=[_ROW_SPEC, _ROW_SPEC, _ROW_SPEC] + _FULL_SPECS,
        out_specs=_ROW_SPEC,
        out_shape=_OUT_SHAPE,
    )(y1, partials[0], partials[1], wt, b2, t2)


@jax.jit
def kernel(features, edge_index, W, b, theta1, theta2):
    src = edge_index[0].astype(jnp.int32).reshape(NW, E_PER_W)
    src_pad = jnp.pad(src, ((0, 0), (0, E_PAD_W - E_PER_W))).reshape(-1)
    dst = edge_index[1].astype(jnp.int32)
    zeros = jnp.zeros((ROWS_PER_TILE, D), jnp.float32)
    wt = W.T
    b2 = b.reshape(1, D)
    t1 = theta1.reshape(1, 1)
    t2 = theta2.reshape(1, 1)
    y1 = _tc_feat(features, wt, b2, t1)
    partials = _sc_scatter_sum(features, src_pad, dst, zeros)
    return _tc_neigh(y1, partials, wt, b2, t2)


# speedup vs baseline: 1.8191x; 1.8191x over previous
import functools

import jax
import jax.numpy as jnp
from jax import lax
from jax.experimental import pallas as pl
from jax.experimental.pallas import tpu as pltpu
from jax.experimental.pallas import tpu_sc as plsc

N_NODES = 10000
N_EDGES = 320000
D = 128

NC = 2
NS = 16
NW = NC * NS
E_PER_W = N_EDGES // NW
CHUNK = 128
FULL_CHUNKS = E_PER_W // CHUNK
TAIL_E = E_PER_W - FULL_CHUNKS * CHUNK
N_CH = FULL_CHUNKS + 1
E_PAD_W = N_CH * CHUNK
NB = 6
NR = 2
ROWS_PER_TILE = 624
TAIL_OFF = ROWS_PER_TILE * NS
TAIL_ROWS = N_NODES - TAIL_OFF


def _sc_scatter_sum(features, src_pad, dst, zeros):
    mesh = plsc.VectorSubcoreMesh(
        core_axis_name="c", subcore_axis_name="s", num_cores=NC, num_subcores=NS
    )

    @functools.partial(
        pl.kernel,
        out_type=jax.ShapeDtypeStruct((NC, N_NODES, D), jnp.float32),
        mesh=mesh,
        scratch_types=[
            pltpu.VMEM_SHARED((N_NODES, D), jnp.float32),
            [pltpu.VMEM((CHUNK,), jnp.int32)] * NB,
            [pltpu.VMEM((CHUNK,), jnp.int32)] * NB,
            pltpu.VMEM((TAIL_E,), jnp.int32),
            pltpu.VMEM((CHUNK, D), jnp.float32),
            pltpu.VMEM((CHUNK, D), jnp.float32),
            [pltpu.SemaphoreType.DMA] * NB,
            [pltpu.SemaphoreType.DMA] * NB,
            pltpu.SemaphoreType.DMA,
            pltpu.SemaphoreType.DMA,
            pltpu.SemaphoreType.DMA,
        ],
    )
    def k(feat_hbm, src_hbm, dst_hbm, zeros_hbm, out_hbm,
          acc, srcb, dstb, dbt, rows0, rows1, ssrc, sdst, sdt, sg0, sg1):
        c = lax.axis_index("c")
        s = lax.axis_index("s")
        wid = s * NC + c
        base_s = wid * E_PAD_W
        base_d = wid * E_PER_W
        rows = (rows0, rows1)
        sg = (sg0, sg1)

        pltpu.sync_copy(zeros_hbm, acc.at[pl.ds(s * ROWS_PER_TILE, ROWS_PER_TILE)])

        @pl.when(s == NS - 1)
        def _():
            pltpu.sync_copy(zeros_hbm.at[pl.ds(0, TAIL_ROWS)],
                            acc.at[pl.ds(TAIL_OFF, TAIL_ROWS)])

        for j in range(NB):
            pltpu.async_copy(src_hbm.at[pl.ds(base_s + j * CHUNK, CHUNK)],
                             srcb[j], ssrc[j])
            pltpu.async_copy(dst_hbm.at[pl.ds(base_d + j * CHUNK, CHUNK)],
                             dstb[j], sdst[j])
        pltpu.async_copy(
            dst_hbm.at[pl.ds(base_d + FULL_CHUNKS * CHUNK, TAIL_E)], dbt, sdt)
        plsc.subcore_barrier()

        idx_bytes = src_hbm.at[pl.ds(base_s, CHUNK)]

        for j in range(NR - 1):
            pltpu.make_async_copy(idx_bytes, srcb[j], ssrc[j]).wait()
            pltpu.async_copy(feat_hbm.at[srcb[j]], rows[j], sg[j])

        def step(i, b, bn, r, rn):
            pltpu.make_async_copy(idx_bytes, srcb[bn], ssrc[bn]).wait()
            pltpu.async_copy(feat_hbm.at[srcb[bn]], rows[rn], sg[rn])
            pltpu.make_async_copy(feat_hbm.at[srcb[b]], rows[r], sg[r]).wait()

            @pl.when(i + NB <= FULL_CHUNKS)
            def _():
                off = base_s + (i + NB) * CHUNK
                pltpu.async_copy(src_hbm.at[pl.ds(off, CHUNK)], srcb[b], ssrc[b])

            pltpu.make_async_copy(idx_bytes, dstb[b], sdst[b]).wait()
            pltpu.sync_copy(rows[r], acc.at[dstb[b]], add=True)

            @pl.when(i + NB < FULL_CHUNKS)
            def _():
                off = base_d + (i + NB) * CHUNK
                pltpu.async_copy(dst_hbm.at[pl.ds(off, CHUNK)], dstb[b], sdst[b])

        def body(t, _):
            i0 = t * NB
            for kk in range(NB):
                step(i0 + kk, kk, (kk + NR - 1) % NB, kk % NR, (kk + NR - 1) % NR)
            return ()

        lax.fori_loop(0, FULL_CHUNKS // NB, body, ())

        rt = FULL_CHUNKS % NR
        bt = FULL_CHUNKS % NB
        pltpu.make_async_copy(feat_hbm.at[srcb[bt]], rows[rt], sg[rt]).wait()
        pltpu.make_async_copy(
            dst_hbm.at[pl.ds(base_d, TAIL_E)], dbt, sdt).wait()
        pltpu.sync_copy(rows[rt].at[pl.ds(0, TAIL_E)], acc.at[dbt], add=True)

        plsc.subcore_barrier()
        pltpu.sync_copy(
            acc.at[pl.ds(s * ROWS_PER_TILE, ROWS_PER_TILE)],
            out_hbm.at[c, pl.ds(s * ROWS_PER_TILE, ROWS_PER_TILE)],
        )

        @pl.when(s == NS - 1)
        def _():
            pltpu.sync_copy(acc.at[pl.ds(TAIL_OFF, TAIL_ROWS)],
                            out_hbm.at[c, pl.ds(TAIL_OFF, TAIL_ROWS)])

    return k(features, src_pad, dst, zeros)


def _tc_feat_body(f_ref, wt_ref, b_ref, t_ref, o_ref):
    a = jnp.dot(f_ref[...], wt_ref[...], preferred_element_type=jnp.float32)
    o_ref[...] = t_ref[0, 0] * jnp.maximum(a + b_ref[...], 0.0)


def _tc_neigh_body(y1_ref, p0_ref, p1_ref, wt_ref, b_ref, t_ref, o_ref):
    hn = p0_ref[...] + p1_ref[...]
    a = jnp.dot(hn, wt_ref[...], preferred_element_type=jnp.float32)
    o_ref[...] = y1_ref[...] + t_ref[0, 0] * jnp.maximum(a + b_ref[...], 0.0)


_ROW_SPEC = pl.BlockSpec((1000, D), lambda i: (i, 0))
_FULL_SPECS = [
    pl.BlockSpec((D, D), lambda i: (0, 0)),
    pl.BlockSpec((1, D), lambda i: (0, 0)),
    pl.BlockSpec(memory_space=pltpu.SMEM),
]
_OUT_SHAPE = jax.ShapeDtypeStruct((N_NODES, D), jnp.float32)


def _tc_feat(features, wt, b2, t1):
    return pl.pallas_call(
        _tc_feat_body,
        grid=(10,),
        in_specs=[_ROW_SPEC] + _FULL_SPECS,
        out_specs=_ROW_SPEC,
        out_shape=_OUT_SHAPE,
    )(features, wt, b2, t1)


def _tc_neigh(y1, partials, wt, b2, t2):
    return pl.pallas_call(
        _tc_neigh_body,
        grid=(10,),
        in_specs=[_ROW_SPEC, _ROW_SPEC, _ROW_SPEC] + _FULL_SPECS,
        out_specs=_ROW_SPEC,
        out_shape=_OUT_SHAPE,
    )(y1, partials[0], partials[1], wt, b2, t2)


@jax.jit
def kernel(features, edge_index, W, b, theta1, theta2):
    src = edge_index[0].astype(jnp.int32).reshape(NW, E_PER_W)
    pad_idx = jnp.broadcast_to(
        jnp.arange(E_PAD_W - E_PER_W, dtype=jnp.int32), (NW, E_PAD_W - E_PER_W))
    src_pad = jnp.concatenate([src, pad_idx], axis=1).reshape(-1)
    dst = edge_index[1].astype(jnp.int32)
    zeros = jnp.zeros((ROWS_PER_TILE, D), jnp.float32)
    wt = W.T
    b2 = b.reshape(1, D)
    t1 = theta1.reshape(1, 1)
    t2 = theta2.reshape(1, 1)
    y1 = _tc_feat(features, wt, b2, t1)
    partials = _sc_scatter_sum(features, src_pad, dst, zeros)
    return _tc_neigh(y1, partials, wt, b2, t2)
